# Initial kernel scaffold; baseline (speedup 1.0000x reference)
import jax
import jax.numpy as jnp
from jax.experimental import pallas as pl


def kernel(x, edge_index, weight, bias):
    raise NotImplementedError("write your pallas kernel here")


# speedup vs baseline: 13.9253x; 13.9253x over previous
import dataclasses
import functools

import jax
import jax.numpy as jnp
from jax import lax
from jax.experimental import pallas as pl
from jax.experimental.pallas import tpu as pltpu
from jax.experimental.pallas import tpu_sc as plsc

N_NODES = 10000
D = 128
E = 320000

NC = 2
NS = 16
NW = NC * NS
EPW = E // NW
CHUNK = 80
NCH = EPW // CHUNK
RPS = N_NODES // NS
HL = 16
NBINS = 10240
NBR = NBINS // 128


def _hist_body(row_hbm, col_hbm, out_hbm, rbins, cbins, ridx, cidx, obuf):
    cid = lax.axis_index("c")
    sid = lax.axis_index("s")
    wid = sid * NC + cid
    base = wid * EPW
    zeros16 = jnp.zeros((HL,), jnp.float32)

    @pl.loop(0, NBINS // HL)
    def _(i):
        rbins[pl.ds(i * HL, HL)] = zeros16
        cbins[pl.ds(i * HL, HL)] = zeros16

    @pl.loop(0, NCH)
    def _(j):
        b = base + j * CHUNK
        pltpu.sync_copy(row_hbm.at[pl.ds(b, CHUNK)], ridx)
        pltpu.sync_copy(col_hbm.at[pl.ds(b, CHUNK)], cidx)
        for v in range(CHUNK // HL):
            k = ridx[pl.ds(v * HL, HL)]
            cnt, last = plsc.scan_count(k)
            plsc.addupdate_scatter(rbins, [k], cnt.astype(jnp.float32),
                                   mask=last)
            k2 = cidx[pl.ds(v * HL, HL)]
            cnt2, last2 = plsc.scan_count(k2)
            plsc.addupdate_scatter(cbins, [k2], cnt2.astype(jnp.float32),
                                   mask=last2)

    def emit(bins, t):
        @pl.loop(0, NBR)
        def _(r):
            for l in range(8):
                obuf[r, pl.ds(l * HL, HL)] = bins[pl.ds((r * 8 + l) * HL, HL)]

        pltpu.sync_copy(obuf, out_hbm.at[cid, sid, t])

    emit(rbins, 0)
    emit(cbins, 1)


def _edge_body(y_hbm, row_hbm, col_hbm, zeros_hbm, out_hbm,
               acc, ridx, cidx, fbuf):
    cid = lax.axis_index("c")
    sid = lax.axis_index("s")
    wid = sid * NC + cid
    base = wid * EPW
    rs = sid * RPS

    pltpu.sync_copy(zeros_hbm, acc.at[pl.ds(rs, RPS)])
    plsc.subcore_barrier()

    @pl.loop(0, NCH)
    def _(j):
        b = base + j * CHUNK
        pltpu.sync_copy(row_hbm.at[pl.ds(b, CHUNK)], ridx)
        pltpu.sync_copy(col_hbm.at[pl.ds(b, CHUNK)], cidx)
        pltpu.sync_copy(y_hbm.at[ridx], fbuf)
        pltpu.sync_copy(fbuf, acc.at[cidx], add=True)

    plsc.subcore_barrier()
    pltpu.sync_copy(acc.at[pl.ds(rs, RPS)], out_hbm.at[cid, sid])


@functools.cache
def _sc_kernels():
    mesh = plsc.VectorSubcoreMesh(core_axis_name="c", subcore_axis_name="s")
    cp = pltpu.CompilerParams()
    if "needs_layout_passes" in pltpu.CompilerParams.__dataclass_fields__:
        cp = dataclasses.replace(cp, needs_layout_passes=False)
    hist = pl.kernel(
        _hist_body,
        out_type=jax.ShapeDtypeStruct((NC, NS, 2, NBR, 128), jnp.float32),
        mesh=mesh,
        compiler_params=cp,
        scratch_types=[
            pltpu.VMEM((NBINS,), jnp.float32),
            pltpu.VMEM((NBINS,), jnp.float32),
            pltpu.VMEM((CHUNK,), jnp.int32),
            pltpu.VMEM((CHUNK,), jnp.int32),
            pltpu.VMEM((NBR, 128), jnp.float32),
        ],
    )
    edge = pl.kernel(
        _edge_body,
        out_type=jax.ShapeDtypeStruct((NC, NS, RPS, D), jnp.float32),
        mesh=mesh,
        scratch_types=[
            pltpu.VMEM_SHARED((N_NODES, D), jnp.float32),
            pltpu.VMEM((CHUNK,), jnp.int32),
            pltpu.VMEM((CHUNK,), jnp.int32),
            pltpu.VMEM((CHUNK, D), jnp.float32),
        ],
    )
    return hist, edge


def _deg_body(h_ref, s_ref):
    deg = jnp.sum(h_ref[...], axis=0)
    s_ref[...] = jnp.where(deg > 0.5, lax.rsqrt(jnp.maximum(deg, 1e-12)), 0.0)


_deg_scale = pl.pallas_call(
    _deg_body,
    out_shape=jax.ShapeDtypeStruct((2, NBR, 128), jnp.float32),
)

_BLK = 2000


def _scale_mm_body(x_ref, w_ref, r_ref, y_ref):
    xw = jnp.dot(x_ref[...], w_ref[...], preferred_element_type=jnp.float32)
    y_ref[...] = xw * r_ref[...]


_scale_mm = pl.pallas_call(
    _scale_mm_body,
    grid=(N_NODES // _BLK,),
    in_specs=[
        pl.BlockSpec((_BLK, D), lambda i: (i, 0)),
        pl.BlockSpec((D, D), lambda i: (0, 0)),
        pl.BlockSpec((_BLK, 1), lambda i: (i, 0)),
    ],
    out_specs=pl.BlockSpec((_BLK, D), lambda i: (i, 0)),
    out_shape=jax.ShapeDtypeStruct((N_NODES, D), jnp.float32),
)


def _finish_body(p_ref, c_ref, b_ref, o_ref):
    o_ref[...] = (p_ref[0] + p_ref[1]) * c_ref[...] + b_ref[...]


_finish = pl.pallas_call(
    _finish_body,
    grid=(N_NODES // _BLK,),
    in_specs=[
        pl.BlockSpec((NC, _BLK, D), lambda i: (0, i, 0)),
        pl.BlockSpec((_BLK, 1), lambda i: (i, 0)),
        pl.BlockSpec((1, D), lambda i: (0, 0)),
    ],
    out_specs=pl.BlockSpec((_BLK, D), lambda i: (i, 0)),
    out_shape=jax.ShapeDtypeStruct((N_NODES, D), jnp.float32),
)


def kernel(x, edge_index, weight, bias):
    row = edge_index[0].astype(jnp.int32)
    col = edge_index[1].astype(jnp.int32)
    zeros_d = jnp.zeros((RPS, D), jnp.float32)

    hist_kernel, edge_kernel = _sc_kernels()
    hist = hist_kernel(row, col).reshape(NW, 2, NBR, 128)
    scale = _deg_scale(hist).reshape(2, NBINS)[:, :N_NODES]
    r_vec = scale[0].reshape(N_NODES, 1)
    c_vec = scale[1].reshape(N_NODES, 1)
    y = _scale_mm(x, weight, r_vec)
    partials = edge_kernel(y, row, col, zeros_d).reshape(NC, N_NODES, D)
    return _finish(partials, c_vec, bias.reshape(1, D))


# speedup vs baseline: 20.0633x; 1.4408x over previous
import dataclasses
import functools

import jax
import jax.numpy as jnp
from jax import lax
from jax.experimental import pallas as pl
from jax.experimental.pallas import tpu as pltpu
from jax.experimental.pallas import tpu_sc as plsc

N_NODES = 10000
D = 128
E = 320000

NC = 2
NS = 16
NW = NC * NS
EPW = E // NW
CHUNK = 80
NCH = EPW // CHUNK
RPS = N_NODES // NS
HL = 16
NBINS = 10240
NBR = NBINS // 128


def _hist_body(row_hbm, col_hbm, out_hbm, rbins, cbins, ridx, cidx, obuf):
    cid = lax.axis_index("c")
    sid = lax.axis_index("s")
    wid = sid * NC + cid
    base = wid * EPW
    zeros16 = jnp.zeros((HL,), jnp.float32)

    @pl.loop(0, NBINS // HL)
    def _(i):
        rbins[pl.ds(i * HL, HL)] = zeros16
        cbins[pl.ds(i * HL, HL)] = zeros16

    pltpu.sync_copy(row_hbm.at[pl.ds(base, EPW)], ridx)
    pltpu.sync_copy(col_hbm.at[pl.ds(base, EPW)], cidx)

    @pl.loop(0, EPW // HL)
    def _(v):
        k = ridx[pl.ds(v * HL, HL)]
        cnt, last = plsc.scan_count(k)
        plsc.addupdate_scatter(rbins, [k], cnt.astype(jnp.float32), mask=last)
        k2 = cidx[pl.ds(v * HL, HL)]
        cnt2, last2 = plsc.scan_count(k2)
        plsc.addupdate_scatter(cbins, [k2], cnt2.astype(jnp.float32),
                               mask=last2)

    def emit(bins, t):
        @pl.loop(0, NBR)
        def _(r):
            for l in range(8):
                obuf[r, pl.ds(l * HL, HL)] = bins[pl.ds((r * 8 + l) * HL, HL)]

        pltpu.sync_copy(obuf, out_hbm.at[cid, sid, t])

    emit(rbins, 0)
    emit(cbins, 1)


def _edge_body(y_hbm, row_hbm, col_hbm, zeros_hbm, out_hbm,
               acc, ridx, cidx, fbuf):
    cid = lax.axis_index("c")
    sid = lax.axis_index("s")
    wid = sid * NC + cid
    base = wid * EPW
    rs = sid * RPS

    pltpu.sync_copy(zeros_hbm, acc.at[pl.ds(rs, RPS)])
    pltpu.sync_copy(row_hbm.at[pl.ds(base, EPW)], ridx)
    plsc.subcore_barrier()

    @pl.loop(0, NCH)
    def _(j):
        b = base + j * CHUNK
        pltpu.sync_copy(col_hbm.at[pl.ds(b, CHUNK)], cidx)
        pltpu.sync_copy(y_hbm.at[ridx.at[pl.ds(j * CHUNK, CHUNK)]], fbuf)
        pltpu.sync_copy(fbuf, acc.at[cidx], add=True)

    plsc.subcore_barrier()
    pltpu.sync_copy(acc.at[pl.ds(rs, RPS)], out_hbm.at[cid, sid])


@functools.cache
def _sc_kernels():
    mesh = plsc.VectorSubcoreMesh(core_axis_name="c", subcore_axis_name="s")
    cp = pltpu.CompilerParams()
    if "needs_layout_passes" in pltpu.CompilerParams.__dataclass_fields__:
        cp = dataclasses.replace(cp, needs_layout_passes=False)
    hist = pl.kernel(
        _hist_body,
        out_type=jax.ShapeDtypeStruct((NC, NS, 2, NBR, 128), jnp.float32),
        mesh=mesh,
        compiler_params=cp,
        scratch_types=[
            pltpu.VMEM((NBINS,), jnp.float32),
            pltpu.VMEM((NBINS,), jnp.float32),
            pltpu.VMEM((EPW,), jnp.int32),
            pltpu.VMEM((EPW,), jnp.int32),
            pltpu.VMEM((NBR, 128), jnp.float32),
        ],
    )
    edge = pl.kernel(
        _edge_body,
        out_type=jax.ShapeDtypeStruct((NC, NS, RPS, D), jnp.float32),
        mesh=mesh,
        scratch_types=[
            pltpu.VMEM_SHARED((N_NODES, D), jnp.float32),
            pltpu.VMEM((EPW,), jnp.int32),
            pltpu.VMEM((CHUNK,), jnp.int32),
            pltpu.VMEM((CHUNK, D), jnp.float32),
        ],
    )
    return hist, edge


def _deg_body(h_ref, s_ref):
    deg = jnp.sum(h_ref[...], axis=0)
    s_ref[...] = jnp.where(deg > 0.5, lax.rsqrt(jnp.maximum(deg, 1e-12)), 0.0)


_deg_scale = pl.pallas_call(
    _deg_body,
    out_shape=jax.ShapeDtypeStruct((2, NBR, 128), jnp.float32),
)

_BLK = 2000


def _mm_body(x_ref, w_ref, y_ref):
    y_ref[...] = jnp.dot(x_ref[...], w_ref[...],
                         preferred_element_type=jnp.float32)


_mm = pl.pallas_call(
    _mm_body,
    grid=(N_NODES // _BLK,),
    in_specs=[
        pl.BlockSpec((_BLK, D), lambda i: (i, 0)),
        pl.BlockSpec((D, D), lambda i: (0, 0)),
    ],
    out_specs=pl.BlockSpec((_BLK, D), lambda i: (i, 0)),
    out_shape=jax.ShapeDtypeStruct((N_NODES, D), jnp.float32),
)


def _scale_body(xw_ref, r_ref, y_ref):
    y_ref[...] = xw_ref[...] * r_ref[...]


_scale = pl.pallas_call(
    _scale_body,
    grid=(N_NODES // _BLK,),
    in_specs=[
        pl.BlockSpec((_BLK, D), lambda i: (i, 0)),
        pl.BlockSpec((_BLK, 1), lambda i: (i, 0)),
    ],
    out_specs=pl.BlockSpec((_BLK, D), lambda i: (i, 0)),
    out_shape=jax.ShapeDtypeStruct((N_NODES, D), jnp.float32),
)


def _finish_body(p_ref, c_ref, b_ref, o_ref):
    o_ref[...] = (p_ref[0] + p_ref[1]) * c_ref[...] + b_ref[...]


_finish = pl.pallas_call(
    _finish_body,
    grid=(N_NODES // _BLK,),
    in_specs=[
        pl.BlockSpec((NC, _BLK, D), lambda i: (0, i, 0)),
        pl.BlockSpec((_BLK, 1), lambda i: (i, 0)),
        pl.BlockSpec((1, D), lambda i: (0, 0)),
    ],
    out_specs=pl.BlockSpec((_BLK, D), lambda i: (i, 0)),
    out_shape=jax.ShapeDtypeStruct((N_NODES, D), jnp.float32),
)


def kernel(x, edge_index, weight, bias):
    row = edge_index[0].astype(jnp.int32)
    col = edge_index[1].astype(jnp.int32)
    zeros_d = jnp.zeros((RPS, D), jnp.float32)

    hist_kernel, edge_kernel = _sc_kernels()
    hist = hist_kernel(row, col).reshape(NW, 2, NBR, 128)
    scale = _deg_scale(hist).reshape(2, NBINS)[:, :N_NODES]
    r_vec = scale[0].reshape(N_NODES, 1)
    c_vec = scale[1].reshape(N_NODES, 1)
    y = _scale(_mm(x, weight), r_vec)
    partials = edge_kernel(y, row, col, zeros_d).reshape(NC, N_NODES, D)
    return _finish(partials, c_vec, bias.reshape(1, D))


# speedup vs baseline: 29.6017x; 1.4754x over previous
import dataclasses
import functools

import jax
import jax.numpy as jnp
from jax import lax
from jax.experimental import pallas as pl
from jax.experimental.pallas import tpu as pltpu
from jax.experimental.pallas import tpu_sc as plsc

N_NODES = 10000
D = 128
E = 320000

NC = 2
NS = 16
NW = NC * NS
EPW = E // NW
CHUNK = 80
NCH = EPW // CHUNK
RPS = N_NODES // NS
HL = 16
NBINS = 10240
NBR = NBINS // 128


def _hist_body(row_hbm, col_hbm, out_hbm, rbins, cbins, ridx, cidx, obuf):
    cid = lax.axis_index("c")
    sid = lax.axis_index("s")
    wid = sid * NC + cid
    base = wid * EPW
    zeros16 = jnp.zeros((HL,), jnp.float32)

    @pl.loop(0, NBINS // HL)
    def _(i):
        rbins[pl.ds(i * HL, HL)] = zeros16
        cbins[pl.ds(i * HL, HL)] = zeros16

    pltpu.sync_copy(row_hbm.at[pl.ds(base, EPW)], ridx)
    pltpu.sync_copy(col_hbm.at[pl.ds(base, EPW)], cidx)

    @pl.loop(0, EPW // HL)
    def _(v):
        k = ridx[pl.ds(v * HL, HL)]
        cnt, last = plsc.scan_count(k)
        plsc.addupdate_scatter(rbins, [k], cnt.astype(jnp.float32), mask=last)
        k2 = cidx[pl.ds(v * HL, HL)]
        cnt2, last2 = plsc.scan_count(k2)
        plsc.addupdate_scatter(cbins, [k2], cnt2.astype(jnp.float32),
                               mask=last2)

    def emit(bins, t):
        @pl.loop(0, NBR)
        def _(r):
            for l in range(8):
                obuf[r, pl.ds(l * HL, HL)] = bins[pl.ds((r * 8 + l) * HL, HL)]

        pltpu.sync_copy(obuf, out_hbm.at[cid, sid, t])

    emit(rbins, 0)
    emit(cbins, 1)


ECH = 128
NFULL = EPW // ECH
NPAIR = NFULL // 2
TAIL = EPW - NFULL * ECH


def _edge_body(y_hbm, row_hbm, col_hbm, zeros_hbm, out_hbm,
               acc, ridx, cidx0, cidx1, cidxt, fbuf0, fbuf1,
               sg0, sg1, sc0, sc1, ss0, ss1):
    cid = lax.axis_index("c")
    sid = lax.axis_index("s")
    wid = sid * NC + cid
    base = wid * EPW
    rs = sid * RPS

    pltpu.sync_copy(zeros_hbm, acc.at[pl.ds(rs, RPS)])
    pltpu.sync_copy(row_hbm.at[pl.ds(base, EPW)], ridx)
    plsc.subcore_barrier()

    bufs = ((cidx0, fbuf0, sg0, sc0, ss0), (cidx1, fbuf1, sg1, sc1, ss1))

    @pl.loop(0, NPAIR)
    def _(q):
        for p in (0, 1):
            j = q * 2 + p
            cidx, fbuf, sg, sc, ss = bufs[p]

            @pl.when(q > 0)
            def _():
                pltpu.make_async_copy(fbuf, acc.at[cidx], ss).wait()

            pltpu.async_copy(col_hbm.at[pl.ds(base + j * ECH, ECH)], cidx, sc)
            pltpu.async_copy(y_hbm.at[ridx.at[pl.ds(j * ECH, ECH)]], fbuf, sg)
        for p in (0, 1):
            cidx, fbuf, sg, sc, ss = bufs[p]
            j = q * 2 + p
            pltpu.make_async_copy(
                y_hbm.at[ridx.at[pl.ds(j * ECH, ECH)]], fbuf, sg).wait()
            pltpu.make_async_copy(
                col_hbm.at[pl.ds(base + j * ECH, ECH)], cidx, sc).wait()
            pltpu.async_copy(fbuf, acc.at[cidx], ss, add=True)

    for p in (0, 1):
        cidx, fbuf, sg, sc, ss = bufs[p]
        pltpu.make_async_copy(fbuf, acc.at[cidx], ss).wait()

    b = base + NFULL * ECH
    pltpu.sync_copy(col_hbm.at[pl.ds(b, TAIL)], cidxt)
    pltpu.sync_copy(y_hbm.at[ridx.at[pl.ds(NFULL * ECH, TAIL)]],
                    fbuf0.at[pl.ds(0, TAIL)])
    pltpu.sync_copy(fbuf0.at[pl.ds(0, TAIL)], acc.at[cidxt], add=True)

    plsc.subcore_barrier()
    pltpu.sync_copy(acc.at[pl.ds(rs, RPS)], out_hbm.at[cid, sid])


@functools.cache
def _sc_kernels():
    mesh = plsc.VectorSubcoreMesh(core_axis_name="c", subcore_axis_name="s")
    cp = pltpu.CompilerParams()
    if "needs_layout_passes" in pltpu.CompilerParams.__dataclass_fields__:
        cp = dataclasses.replace(cp, needs_layout_passes=False)
    hist = pl.kernel(
        _hist_body,
        out_type=jax.ShapeDtypeStruct((NC, NS, 2, NBR, 128), jnp.float32),
        mesh=mesh,
        compiler_params=cp,
        scratch_types=[
            pltpu.VMEM((NBINS,), jnp.float32),
            pltpu.VMEM((NBINS,), jnp.float32),
            pltpu.VMEM((EPW,), jnp.int32),
            pltpu.VMEM((EPW,), jnp.int32),
            pltpu.VMEM((NBR, 128), jnp.float32),
        ],
    )
    edge = pl.kernel(
        _edge_body,
        out_type=jax.ShapeDtypeStruct((NC, NS, RPS, D), jnp.float32),
        mesh=mesh,
        scratch_types=[
            pltpu.VMEM_SHARED((N_NODES, D), jnp.float32),
            pltpu.VMEM((EPW,), jnp.int32),
            pltpu.VMEM((ECH,), jnp.int32),
            pltpu.VMEM((ECH,), jnp.int32),
            pltpu.VMEM((TAIL,), jnp.int32),
            pltpu.VMEM((ECH, D), jnp.float32),
            pltpu.VMEM((ECH, D), jnp.float32),
            pltpu.SemaphoreType.DMA,
            pltpu.SemaphoreType.DMA,
            pltpu.SemaphoreType.DMA,
            pltpu.SemaphoreType.DMA,
            pltpu.SemaphoreType.DMA,
            pltpu.SemaphoreType.DMA,
        ],
    )
    return hist, edge


def _deg_body(h_ref, s_ref):
    deg = jnp.sum(h_ref[...], axis=0)
    s_ref[...] = jnp.where(deg > 0.5, lax.rsqrt(jnp.maximum(deg, 1e-12)), 0.0)


_deg_scale = pl.pallas_call(
    _deg_body,
    out_shape=jax.ShapeDtypeStruct((2, NBR, 128), jnp.float32),
)

_BLK = 2000


def _mm_body(x_ref, w_ref, y_ref):
    y_ref[...] = jnp.dot(x_ref[...], w_ref[...],
                         preferred_element_type=jnp.float32)


_mm = pl.pallas_call(
    _mm_body,
    grid=(N_NODES // _BLK,),
    in_specs=[
        pl.BlockSpec((_BLK, D), lambda i: (i, 0)),
        pl.BlockSpec((D, D), lambda i: (0, 0)),
    ],
    out_specs=pl.BlockSpec((_BLK, D), lambda i: (i, 0)),
    out_shape=jax.ShapeDtypeStruct((N_NODES, D), jnp.float32),
)


def _scale_body(xw_ref, r_ref, y_ref):
    y_ref[...] = xw_ref[...] * r_ref[...]


_scale = pl.pallas_call(
    _scale_body,
    grid=(N_NODES // _BLK,),
    in_specs=[
        pl.BlockSpec((_BLK, D), lambda i: (i, 0)),
        pl.BlockSpec((_BLK, 1), lambda i: (i, 0)),
    ],
    out_specs=pl.BlockSpec((_BLK, D), lambda i: (i, 0)),
    out_shape=jax.ShapeDtypeStruct((N_NODES, D), jnp.float32),
)


def _finish_body(p_ref, c_ref, b_ref, o_ref):
    o_ref[...] = (p_ref[0] + p_ref[1]) * c_ref[...] + b_ref[...]


_finish = pl.pallas_call(
    _finish_body,
    grid=(N_NODES // _BLK,),
    in_specs=[
        pl.BlockSpec((NC, _BLK, D), lambda i: (0, i, 0)),
        pl.BlockSpec((_BLK, 1), lambda i: (i, 0)),
        pl.BlockSpec((1, D), lambda i: (0, 0)),
    ],
    out_specs=pl.BlockSpec((_BLK, D), lambda i: (i, 0)),
    out_shape=jax.ShapeDtypeStruct((N_NODES, D), jnp.float32),
)


def kernel(x, edge_index, weight, bias):
    row = edge_index[0].astype(jnp.int32)
    col = edge_index[1].astype(jnp.int32)
    zeros_d = jnp.zeros((RPS, D), jnp.float32)

    hist_kernel, edge_kernel = _sc_kernels()
    hist = hist_kernel(row, col).reshape(NW, 2, NBR, 128)
    scale = _deg_scale(hist).reshape(2, NBINS)[:, :N_NODES]
    r_vec = scale[0].reshape(N_NODES, 1)
    c_vec = scale[1].reshape(N_NODES, 1)
    y = _scale(_mm(x, weight), r_vec)
    partials = edge_kernel(y, row, col, zeros_d).reshape(NC, N_NODES, D)
    return _finish(partials, c_vec, bias.reshape(1, D))


# speedup vs baseline: 29.7639x; 1.0055x over previous
import dataclasses
import functools

import jax
import jax.numpy as jnp
from jax import lax
from jax.experimental import pallas as pl
from jax.experimental.pallas import tpu as pltpu
from jax.experimental.pallas import tpu_sc as plsc

N_NODES = 10000
D = 128
E = 320000

NC = 2
NS = 16
NW = NC * NS
EPW = E // NW
CHUNK = 80
NCH = EPW // CHUNK
RPS = N_NODES // NS
HL = 16
NBINS = 10240
NBR = NBINS // 128


def _hist_body(row_hbm, col_hbm, out_hbm, rbins, cbins, ridx, cidx, obuf):
    cid = lax.axis_index("c")
    sid = lax.axis_index("s")
    wid = sid * NC + cid
    base = wid * EPW
    zeros16 = jnp.zeros((HL,), jnp.float32)

    @pl.loop(0, NBINS // HL)
    def _(i):
        rbins[pl.ds(i * HL, HL)] = zeros16
        cbins[pl.ds(i * HL, HL)] = zeros16

    pltpu.sync_copy(row_hbm.at[pl.ds(base, EPW)], ridx)
    pltpu.sync_copy(col_hbm.at[pl.ds(base, EPW)], cidx)

    @pl.loop(0, EPW // HL)
    def _(v):
        k = ridx[pl.ds(v * HL, HL)]
        cnt, last = plsc.scan_count(k)
        plsc.addupdate_scatter(rbins, [k], cnt.astype(jnp.float32), mask=last)
        k2 = cidx[pl.ds(v * HL, HL)]
        cnt2, last2 = plsc.scan_count(k2)
        plsc.addupdate_scatter(cbins, [k2], cnt2.astype(jnp.float32),
                               mask=last2)

    def emit(bins, t):
        @pl.loop(0, NBR)
        def _(r):
            for l in range(8):
                obuf[r, pl.ds(l * HL, HL)] = bins[pl.ds((r * 8 + l) * HL, HL)]

        pltpu.sync_copy(obuf, out_hbm.at[cid, sid, t])

    emit(rbins, 0)
    emit(cbins, 1)


ECH = 128
NFULL = EPW // ECH
NPAIR = NFULL // 2
TAIL = EPW - NFULL * ECH


def _edge_body(y_hbm, row_hbm, col_hbm, zeros_hbm, out_hbm,
               acc, ridx, cidx0, cidx1, cidxt, fbuf0, fbuf1,
               sg0, sg1, sc0, sc1, ss0, ss1):
    cid = lax.axis_index("c")
    sid = lax.axis_index("s")
    wid = sid * NC + cid
    base = wid * EPW
    rs = sid * RPS

    pltpu.sync_copy(zeros_hbm, acc.at[pl.ds(rs, RPS)])
    pltpu.sync_copy(row_hbm.at[pl.ds(base, EPW)], ridx)
    plsc.subcore_barrier()

    bufs = ((cidx0, fbuf0, sg0, sc0, ss0), (cidx1, fbuf1, sg1, sc1, ss1))

    @pl.loop(0, NPAIR)
    def _(q):
        for p in (0, 1):
            j = q * 2 + p
            cidx, fbuf, sg, sc, ss = bufs[p]

            @pl.when(q > 0)
            def _():
                pltpu.make_async_copy(fbuf, acc.at[cidx], ss).wait()

            pltpu.async_copy(col_hbm.at[pl.ds(base + j * ECH, ECH)], cidx, sc)
            pltpu.async_copy(y_hbm.at[ridx.at[pl.ds(j * ECH, ECH)]], fbuf, sg)
        for p in (0, 1):
            cidx, fbuf, sg, sc, ss = bufs[p]
            j = q * 2 + p
            pltpu.make_async_copy(
                y_hbm.at[ridx.at[pl.ds(j * ECH, ECH)]], fbuf, sg).wait()
            pltpu.make_async_copy(
                col_hbm.at[pl.ds(base + j * ECH, ECH)], cidx, sc).wait()
            pltpu.async_copy(fbuf, acc.at[cidx], ss, add=True)

    for p in (0, 1):
        cidx, fbuf, sg, sc, ss = bufs[p]
        pltpu.make_async_copy(fbuf, acc.at[cidx], ss).wait()

    b = base + NFULL * ECH
    pltpu.sync_copy(col_hbm.at[pl.ds(b, TAIL)], cidxt)
    pltpu.sync_copy(y_hbm.at[ridx.at[pl.ds(NFULL * ECH, TAIL)]],
                    fbuf0.at[pl.ds(0, TAIL)])
    pltpu.sync_copy(fbuf0.at[pl.ds(0, TAIL)], acc.at[cidxt], add=True)

    plsc.subcore_barrier()
    pltpu.sync_copy(acc.at[pl.ds(rs, RPS)], out_hbm.at[cid, sid])


@functools.cache
def _sc_kernels():
    mesh = plsc.VectorSubcoreMesh(core_axis_name="c", subcore_axis_name="s")
    cp = pltpu.CompilerParams()
    if "needs_layout_passes" in pltpu.CompilerParams.__dataclass_fields__:
        cp = dataclasses.replace(cp, needs_layout_passes=False)
    hist = pl.kernel(
        _hist_body,
        out_type=jax.ShapeDtypeStruct((NC, NS, 2, NBR, 128), jnp.float32),
        mesh=mesh,
        compiler_params=cp,
        scratch_types=[
            pltpu.VMEM((NBINS,), jnp.float32),
            pltpu.VMEM((NBINS,), jnp.float32),
            pltpu.VMEM((EPW,), jnp.int32),
            pltpu.VMEM((EPW,), jnp.int32),
            pltpu.VMEM((NBR, 128), jnp.float32),
        ],
    )
    edge = pl.kernel(
        _edge_body,
        out_type=jax.ShapeDtypeStruct((NC, NS, RPS, D), jnp.float32),
        mesh=mesh,
        scratch_types=[
            pltpu.VMEM_SHARED((N_NODES, D), jnp.float32),
            pltpu.VMEM((EPW,), jnp.int32),
            pltpu.VMEM((ECH,), jnp.int32),
            pltpu.VMEM((ECH,), jnp.int32),
            pltpu.VMEM((TAIL,), jnp.int32),
            pltpu.VMEM((ECH, D), jnp.float32),
            pltpu.VMEM((ECH, D), jnp.float32),
            pltpu.SemaphoreType.DMA,
            pltpu.SemaphoreType.DMA,
            pltpu.SemaphoreType.DMA,
            pltpu.SemaphoreType.DMA,
            pltpu.SemaphoreType.DMA,
            pltpu.SemaphoreType.DMA,
        ],
    )
    return hist, edge


_BLK = 2000


def _deg_body(h_ref, s_ref):
    deg = jnp.sum(h_ref[...], axis=0)
    s_ref[...] = jnp.where(deg > 0.5, lax.rsqrt(jnp.maximum(deg, 1e-12)), 0.0)


_deg_scale = pl.pallas_call(
    _deg_body,
    out_shape=jax.ShapeDtypeStruct((2, NBR, 128), jnp.float32),
)


def _mm_scale_body(x_ref, w_ref, r_ref, y_ref):
    xw = jnp.dot(x_ref[...], w_ref[...], preferred_element_type=jnp.float32)
    y_ref[...] = xw * r_ref[...]


_mm_scale = pl.pallas_call(
    _mm_scale_body,
    grid=(N_NODES // _BLK,),
    in_specs=[
        pl.BlockSpec((_BLK, D), lambda i: (i, 0)),
        pl.BlockSpec((D, D), lambda i: (0, 0)),
        pl.BlockSpec((_BLK, 1), lambda i: (i, 0)),
    ],
    out_specs=pl.BlockSpec((_BLK, D), lambda i: (i, 0)),
    out_shape=jax.ShapeDtypeStruct((N_NODES, D), jnp.float32),
)


def _finish_body(p_ref, c_ref, b_ref, o_ref):
    o_ref[...] = (p_ref[0] + p_ref[1]) * c_ref[...] + b_ref[...]


_finish = pl.pallas_call(
    _finish_body,
    grid=(N_NODES // _BLK,),
    in_specs=[
        pl.BlockSpec((NC, _BLK, D), lambda i: (0, i, 0)),
        pl.BlockSpec((_BLK, 1), lambda i: (i, 0)),
        pl.BlockSpec((1, D), lambda i: (0, 0)),
    ],
    out_specs=pl.BlockSpec((_BLK, D), lambda i: (i, 0)),
    out_shape=jax.ShapeDtypeStruct((N_NODES, D), jnp.float32),
)


def kernel(x, edge_index, weight, bias):
    row = edge_index[0].astype(jnp.int32)
    col = edge_index[1].astype(jnp.int32)
    zeros_d = jnp.zeros((RPS, D), jnp.float32)

    hist_kernel, edge_kernel = _sc_kernels()
    hist = hist_kernel(row, col).reshape(NW, 2, NBR, 128)
    scale = _deg_scale(hist).reshape(2, NBINS)[:, :N_NODES]
    r_vec = scale[0].reshape(N_NODES, 1)
    c_vec = scale[1].reshape(N_NODES, 1)
    y = _mm_scale(x, weight, r_vec)
    partials = edge_kernel(y, row, col, zeros_d).reshape(NC, N_NODES, D)
    return _finish(partials, c_vec, bias.reshape(1, D))


# speedup vs baseline: 33.2559x; 1.1173x over previous
import dataclasses
import functools

import jax
import jax.numpy as jnp
from jax import lax
from jax.experimental import pallas as pl
from jax.experimental.pallas import tpu as pltpu
from jax.experimental.pallas import tpu_sc as plsc

N_NODES = 10000
D = 128
E = 320000

NC = 2
NS = 16
NW = NC * NS
EPW = E // NW
CHUNK = 80
NCH = EPW // CHUNK
RPS = N_NODES // NS
HL = 16
NBINS = 10240
NBR = NBINS // 128


def _hist_body(row_hbm, col_hbm, out_hbm, rbins, cbins, ridx, cidx, obuf):
    cid = lax.axis_index("c")
    sid = lax.axis_index("s")
    wid = sid * NC + cid
    base = wid * EPW
    zeros16 = jnp.zeros((HL,), jnp.float32)

    @pl.loop(0, NBINS // HL)
    def _(i):
        rbins[pl.ds(i * HL, HL)] = zeros16
        cbins[pl.ds(i * HL, HL)] = zeros16

    pltpu.sync_copy(row_hbm.at[pl.ds(base, EPW)], ridx)
    pltpu.sync_copy(col_hbm.at[pl.ds(base, EPW)], cidx)

    @pl.loop(0, EPW // HL)
    def _(v):
        k = ridx[pl.ds(v * HL, HL)]
        cnt, last = plsc.scan_count(k)
        plsc.addupdate_scatter(rbins, [k], cnt.astype(jnp.float32), mask=last)
        k2 = cidx[pl.ds(v * HL, HL)]
        cnt2, last2 = plsc.scan_count(k2)
        plsc.addupdate_scatter(cbins, [k2], cnt2.astype(jnp.float32),
                               mask=last2)

    def emit(bins, t):
        @pl.loop(0, NBR)
        def _(r):
            for l in range(8):
                obuf[r, pl.ds(l * HL, HL)] = bins[pl.ds((r * 8 + l) * HL, HL)]

        pltpu.sync_copy(obuf, out_hbm.at[cid, sid, t])

    emit(rbins, 0)
    emit(cbins, 1)


ECH = 104
NBUF = 3
NFULL = EPW // ECH
NGRP = NFULL // NBUF
TAIL = EPW - NFULL * ECH


def _edge_body(y_hbm, row_hbm, col_hbm, zeros_hbm, out_hbm,
               acc, ridx, cidxt,
               cidx0, cidx1, cidx2, fbuf0, fbuf1, fbuf2,
               sg0, sg1, sg2, sc0, sc1, sc2, ss0, ss1, ss2):
    cid = lax.axis_index("c")
    sid = lax.axis_index("s")
    wid = sid * NC + cid
    base = wid * EPW
    rs = sid * RPS

    pltpu.sync_copy(zeros_hbm, acc.at[pl.ds(rs, RPS)])
    pltpu.sync_copy(row_hbm.at[pl.ds(base, EPW)], ridx)
    plsc.subcore_barrier()

    bufs = ((cidx0, fbuf0, sg0, sc0, ss0), (cidx1, fbuf1, sg1, sc1, ss1),
            (cidx2, fbuf2, sg2, sc2, ss2))

    @pl.loop(0, NGRP)
    def _(q):
        for p in range(NBUF):
            j = q * NBUF + p
            cidx, fbuf, sg, sc, ss = bufs[p]

            @pl.when(q > 0)
            def _():
                pltpu.make_async_copy(fbuf, acc.at[cidx], ss).wait()

            pltpu.async_copy(col_hbm.at[pl.ds(base + j * ECH, ECH)], cidx, sc)
            pltpu.async_copy(y_hbm.at[ridx.at[pl.ds(j * ECH, ECH)]], fbuf, sg)
        for p in range(NBUF):
            cidx, fbuf, sg, sc, ss = bufs[p]
            j = q * NBUF + p
            pltpu.make_async_copy(
                y_hbm.at[ridx.at[pl.ds(j * ECH, ECH)]], fbuf, sg).wait()
            pltpu.make_async_copy(
                col_hbm.at[pl.ds(base + j * ECH, ECH)], cidx, sc).wait()
            pltpu.async_copy(fbuf, acc.at[cidx], ss, add=True)

    for p in range(NBUF):
        cidx, fbuf, sg, sc, ss = bufs[p]
        pltpu.make_async_copy(fbuf, acc.at[cidx], ss).wait()

    b = base + NFULL * ECH
    pltpu.sync_copy(col_hbm.at[pl.ds(b, TAIL)], cidxt)
    pltpu.sync_copy(y_hbm.at[ridx.at[pl.ds(NFULL * ECH, TAIL)]],
                    fbuf0.at[pl.ds(0, TAIL)])
    pltpu.sync_copy(fbuf0.at[pl.ds(0, TAIL)], acc.at[cidxt], add=True)

    plsc.subcore_barrier()
    pltpu.sync_copy(acc.at[pl.ds(rs, RPS)], out_hbm.at[cid, sid])


@functools.cache
def _sc_kernels():
    mesh = plsc.VectorSubcoreMesh(core_axis_name="c", subcore_axis_name="s")
    cp = pltpu.CompilerParams()
    if "needs_layout_passes" in pltpu.CompilerParams.__dataclass_fields__:
        cp = dataclasses.replace(cp, needs_layout_passes=False)
    hist = pl.kernel(
        _hist_body,
        out_type=jax.ShapeDtypeStruct((NC, NS, 2, NBR, 128), jnp.float32),
        mesh=mesh,
        compiler_params=cp,
        scratch_types=[
            pltpu.VMEM((NBINS,), jnp.float32),
            pltpu.VMEM((NBINS,), jnp.float32),
            pltpu.VMEM((EPW,), jnp.int32),
            pltpu.VMEM((EPW,), jnp.int32),
            pltpu.VMEM((NBR, 128), jnp.float32),
        ],
    )
    edge = pl.kernel(
        _edge_body,
        out_type=jax.ShapeDtypeStruct((NC, NS, RPS, D), jnp.float32),
        mesh=mesh,
        scratch_types=(
            [pltpu.VMEM_SHARED((N_NODES, D), jnp.float32),
             pltpu.VMEM((EPW,), jnp.int32),
             pltpu.VMEM((TAIL,), jnp.int32)]
            + [pltpu.VMEM((ECH,), jnp.int32)] * NBUF
            + [pltpu.VMEM((ECH, D), jnp.float32)] * NBUF
            + [pltpu.SemaphoreType.DMA] * (3 * NBUF)
        ),
    )
    return hist, edge


_BLK = 2000


def _deg_body(h_ref, s_ref):
    deg = jnp.sum(h_ref[...], axis=0)
    s_ref[...] = jnp.where(deg > 0.5, lax.rsqrt(jnp.maximum(deg, 1e-12)), 0.0)


_deg_scale = pl.pallas_call(
    _deg_body,
    out_shape=jax.ShapeDtypeStruct((2, NBR, 128), jnp.float32),
)


def _mm_scale_body(x_ref, w_ref, r_ref, y_ref):
    xw = jnp.dot(x_ref[...], w_ref[...], preferred_element_type=jnp.float32)
    y_ref[...] = xw * r_ref[...]


_mm_scale = pl.pallas_call(
    _mm_scale_body,
    grid=(N_NODES // _BLK,),
    in_specs=[
        pl.BlockSpec((_BLK, D), lambda i: (i, 0)),
        pl.BlockSpec((D, D), lambda i: (0, 0)),
        pl.BlockSpec((_BLK, 1), lambda i: (i, 0)),
    ],
    out_specs=pl.BlockSpec((_BLK, D), lambda i: (i, 0)),
    out_shape=jax.ShapeDtypeStruct((N_NODES, D), jnp.float32),
)


def _finish_body(p_ref, c_ref, b_ref, o_ref):
    o_ref[...] = (p_ref[0] + p_ref[1]) * c_ref[...] + b_ref[...]


_finish = pl.pallas_call(
    _finish_body,
    grid=(N_NODES // _BLK,),
    in_specs=[
        pl.BlockSpec((NC, _BLK, D), lambda i: (0, i, 0)),
        pl.BlockSpec((_BLK, 1), lambda i: (i, 0)),
        pl.BlockSpec((1, D), lambda i: (0, 0)),
    ],
    out_specs=pl.BlockSpec((_BLK, D), lambda i: (i, 0)),
    out_shape=jax.ShapeDtypeStruct((N_NODES, D), jnp.float32),
)


def kernel(x, edge_index, weight, bias):
    row = edge_index[0].astype(jnp.int32)
    col = edge_index[1].astype(jnp.int32)
    zeros_d = jnp.zeros((RPS, D), jnp.float32)

    hist_kernel, edge_kernel = _sc_kernels()
    hist = hist_kernel(row, col).reshape(NW, 2, NBR, 128)
    scale = _deg_scale(hist).reshape(2, NBINS)[:, :N_NODES]
    r_vec = scale[0].reshape(N_NODES, 1)
    c_vec = scale[1].reshape(N_NODES, 1)
    y = _mm_scale(x, weight, r_vec)
    partials = edge_kernel(y, row, col, zeros_d).reshape(NC, N_NODES, D)
    return _finish(partials, c_vec, bias.reshape(1, D))
